# baseline (speedup 1.0000x reference)
import functools

import jax
import jax.numpy as jnp
from jax import lax
from jax.experimental import pallas as pl
from jax.experimental.pallas import tpu as pltpu
from jax.experimental.pallas import tpu_sc as plsc

_N = 8388608
_TOTAL = 4 * _N
_NC = 2
_NS = 16
_L = 16
_NW = _NC * _NS
_WPW = _TOTAL // _NW
_BUF = 32768
_NITER = _WPW // _BUF
_STEPS = _BUF // (4 * _L)
_EPB = _BUF // 4
_EPW = _N // _NW

_mesh = plsc.VectorSubcoreMesh(core_axis_name="c", subcore_axis_name="s")


@functools.partial(
    pl.kernel,
    mesh=_mesh,
    compiler_params=pltpu.CompilerParams(needs_layout_passes=False),
    out_type=[
        jax.ShapeDtypeStruct((_NW, _L), jnp.float32),
        jax.ShapeDtypeStruct((_NW, _L), jnp.int32),
    ],
    scratch_types=[
        pltpu.VMEM((_BUF,), jnp.float32),
        pltpu.VMEM((_BUF,), jnp.float32),
        pltpu.VMEM((_L,), jnp.float32),
        pltpu.VMEM((_L,), jnp.int32),
        pltpu.SemaphoreType.DMA,
        pltpu.SemaphoreType.DMA,
    ],
)
def _phase1(x_hbm, minv_hbm, mini_hbm, buf0, buf1, vout, iout, sem0, sem1):
    wid = lax.axis_index("s") * _NC + lax.axis_index("c")
    wbase = wid * _WPW
    bufs = (buf0, buf1)
    sems = (sem0, sem1)
    lane = lax.iota(jnp.int32, _L)
    gidx = lane * 4

    for b in range(2):
        pltpu.async_copy(
            x_hbm.at[pl.ds(wbase + b * _BUF, _BUF)], bufs[b], sems[b]
        )

    inf = jnp.full((_L,), jnp.inf, jnp.float32)
    zero = jnp.zeros((_L,), jnp.int32)

    def outer(g, carry):
        run_min, run_idx = carry
        for b in range(2):
            it = g * 2 + b
            pltpu.make_async_copy(
                x_hbm.at[pl.ds(0, _BUF)], bufs[b], sems[b]
            ).wait()
            ebase = wid * _EPW + it * _EPB

            def inner(j, c, b=b):
                rm, ri, ci = c
                base = j * (4 * _L)
                idx = base + gidx
                re = plsc.load_gather(bufs[b], [idx])
                im = plsc.load_gather(bufs[b], [idx + 1])
                m = jnp.abs(re) + jnp.abs(im)
                pred = m < rm
                rm = jnp.where(pred, m, rm)
                ri = jnp.where(pred, ci, ri)
                return rm, ri, ci + _L

            ci0 = ebase + lane
            run_min, run_idx, _ = lax.fori_loop(
                0, _STEPS, inner, (run_min, run_idx, ci0)
            )

            nxt = it + 2

            @pl.when(nxt < _NITER)
            def _(b=b, nxt=nxt):
                pltpu.async_copy(
                    x_hbm.at[pl.ds(wbase + nxt * _BUF, _BUF)], bufs[b], sems[b]
                )

        return run_min, run_idx

    run_min, run_idx = lax.fori_loop(0, _NITER // 2, outer, (inf, zero))
    vout[...] = run_min
    iout[...] = run_idx
    pltpu.sync_copy(vout, minv_hbm.at[wid])
    pltpu.sync_copy(iout, mini_hbm.at[wid])


@functools.partial(
    pl.kernel,
    mesh=_mesh,
    compiler_params=pltpu.CompilerParams(needs_layout_passes=False),
    out_type=jax.ShapeDtypeStruct((_L,), jnp.int32),
    scratch_types=[
        pltpu.VMEM((_NW, _L), jnp.float32),
        pltpu.VMEM((_NW, _L), jnp.int32),
        pltpu.VMEM((_L,), jnp.int32),
    ],
)
def _phase2(minv_hbm, mini_hbm, out_hbm, vals, idxs, obuf):
    wid = lax.axis_index("s") * _NC + lax.axis_index("c")

    @pl.when(wid == 0)
    def _():
        pltpu.sync_copy(minv_hbm, vals)
        pltpu.sync_copy(mini_hbm, idxs)

        def body(k, c):
            rm, ri = c
            v = vals[k]
            i = idxs[k]
            pred = v < rm
            return jnp.where(pred, v, rm), jnp.where(pred, i, ri)

        rm, ri = lax.fori_loop(1, _NW, body, (vals[0], idxs[0]))
        mv = jnp.min(rm)
        cand = jnp.where(rm == mv, ri, jnp.int32(2147483647))
        best = jnp.min(cand)
        obuf[...] = jnp.broadcast_to(best * 2 + 1, (_L,))
        pltpu.sync_copy(obuf, out_hbm)


def kernel(x):
    flat = x.reshape(_TOTAL)
    minv, mini = _phase1(flat)
    out = _phase2(minv, mini)
    return out[0]


# speedup vs baseline: 264.1975x; 264.1975x over previous
import functools

import jax
import jax.numpy as jnp
from jax import lax
from jax.experimental import pallas as pl
from jax.experimental.pallas import tpu as pltpu
from jax.experimental.pallas import tpu_sc as plsc

_N = 8388608
_TOTAL = 4 * _N
_NC = 2
_NS = 16
_L = 16
_NW = _NC * _NS
_WPW = _TOTAL // _NW
_BUF = 32768
_NITER = _WPW // _BUF
_BLKS = _BUF // 256
_EPB = _BUF // 4
_EPW = _N // _NW

_mesh = plsc.VectorSubcoreMesh(core_axis_name="c", subcore_axis_name="s")


@functools.partial(
    pl.kernel,
    mesh=_mesh,
    compiler_params=pltpu.CompilerParams(needs_layout_passes=False),
    out_type=[
        jax.ShapeDtypeStruct((_NW, _L), jnp.float32),
        jax.ShapeDtypeStruct((_NW, _L), jnp.int32),
    ],
    scratch_types=[
        pltpu.VMEM((_BUF,), jnp.float32),
        pltpu.VMEM((_BUF,), jnp.float32),
        pltpu.VMEM((_L,), jnp.float32),
        pltpu.VMEM((_L,), jnp.int32),
        pltpu.SemaphoreType.DMA,
        pltpu.SemaphoreType.DMA,
    ],
)
def _phase1(x_hbm, minv_hbm, mini_hbm, buf0, buf1, vout, iout, sem0, sem1):
    wid = lax.axis_index("s") * _NC + lax.axis_index("c")
    wbase = wid * _WPW
    bufs = (buf0, buf1)
    sems = (sem0, sem1)
    lane = lax.iota(jnp.int32, _L)
    iv_re = [32 * a + 2 * lane for a in range(4)]
    iv_im = [32 * a + 128 + 2 * lane for a in range(4)]
    cv = [16 * a + lane for a in range(4)]

    for b in range(2):
        pltpu.async_copy(
            x_hbm.at[pl.ds(wbase + b * _BUF, _BUF)], bufs[b], sems[b]
        )

    inf = jnp.full((_L,), jnp.inf, jnp.float32)
    zero = jnp.zeros((_L,), jnp.int32)

    def outer(g, carry):
        run_min, run_idx = carry
        for b in range(2):
            it = g * 2 + b
            pltpu.make_async_copy(
                x_hbm.at[pl.ds(0, _BUF)], bufs[b], sems[b]
            ).wait()
            ebase = wid * _EPW + it * _EPB

            def inner(j, c, b=b, ebase=ebase):
                rm, ri = c
                wb = j * 256
                cb = ebase + j * 64
                for a in range(4):
                    re = plsc.load_gather(bufs[b], [wb + iv_re[a]])
                    im = plsc.load_gather(bufs[b], [wb + iv_im[a]])
                    m = jnp.abs(re) + jnp.abs(im)
                    ci = cb + cv[a]
                    pred = m < rm
                    rm = jnp.where(pred, m, rm)
                    ri = jnp.where(pred, ci, ri)
                return rm, ri

            run_min, run_idx = lax.fori_loop(
                0, _BLKS, inner, (run_min, run_idx)
            )

            nxt = it + 2

            @pl.when(nxt < _NITER)
            def _(b=b, nxt=nxt):
                pltpu.async_copy(
                    x_hbm.at[pl.ds(wbase + nxt * _BUF, _BUF)], bufs[b], sems[b]
                )

        return run_min, run_idx

    run_min, run_idx = lax.fori_loop(0, _NITER // 2, outer, (inf, zero))
    vout[...] = run_min
    iout[...] = run_idx
    pltpu.sync_copy(vout, minv_hbm.at[wid])
    pltpu.sync_copy(iout, mini_hbm.at[wid])


@functools.partial(
    pl.kernel,
    mesh=_mesh,
    compiler_params=pltpu.CompilerParams(needs_layout_passes=False),
    out_type=jax.ShapeDtypeStruct((_L,), jnp.int32),
    scratch_types=[
        pltpu.VMEM((_NW, _L), jnp.float32),
        pltpu.VMEM((_NW, _L), jnp.int32),
        pltpu.VMEM((_L,), jnp.int32),
    ],
)
def _phase2(minv_hbm, mini_hbm, out_hbm, vals, idxs, obuf):
    wid = lax.axis_index("s") * _NC + lax.axis_index("c")

    @pl.when(wid == 0)
    def _():
        pltpu.sync_copy(minv_hbm, vals)
        pltpu.sync_copy(mini_hbm, idxs)

        def body(k, c):
            rm, ri = c
            v = vals[k]
            i = idxs[k]
            pred = v < rm
            return jnp.where(pred, v, rm), jnp.where(pred, i, ri)

        rm, ri = lax.fori_loop(1, _NW, body, (vals[0], idxs[0]))
        mv = jnp.min(rm)
        cand = jnp.where(rm == mv, ri, jnp.int32(2147483647))
        best = jnp.min(cand)
        obuf[...] = jnp.broadcast_to(best * 2 + 1, (_L,))
        pltpu.sync_copy(obuf, out_hbm)


def kernel(x):
    flat = x.reshape(_TOTAL // 256, 128, 2).transpose(0, 2, 1).reshape(_TOTAL)
    minv, mini = _phase1(flat)
    out = _phase2(minv, mini)
    return out[0]
